# Initial kernel scaffold; baseline (speedup 1.0000x reference)
import jax
import jax.numpy as jnp
from jax.experimental import pallas as pl


def kernel(features, edge_index, W1, b1, W2, b2):
    raise NotImplementedError("write your pallas kernel here")


# speedup vs baseline: 3.5813x; 3.5813x over previous
import functools

import jax
import jax.numpy as jnp
from jax import lax
from jax.experimental import pallas as pl
from jax.experimental.pallas import tpu as pltpu
from jax.experimental.pallas import tpu_sc as plsc

_N = 10000
_E = 320000
_LANES = 128
_NW = 32
_IDXC = 16
_RPW = 80
_NROWS = _NW * _RPW
_NH = 10112
_RPS = _NH // 16

_mesh = plsc.VectorSubcoreMesh(core_axis_name="c", subcore_axis_name="s")


@functools.partial(
    pl.kernel,
    out_type=jax.ShapeDtypeStruct((2, 2, _NH, 16), jnp.float32),
    mesh=_mesh,
    scratch_types=[
        pltpu.VMEM((_IDXC, _LANES), jnp.int32),
        pltpu.VMEM((_LANES, 16), jnp.float32),
        pltpu.VMEM_SHARED((_NH, 16), jnp.float32),
        pltpu.VMEM_SHARED((_NH, 16), jnp.float32),
    ],
)
def _deg_kernel(src_hbm, dst_hbm, ones_hbm, zeros_hbm, out_hbm,
                idx_v, ones_v, hist_src, hist_dst):
    c = lax.axis_index("c")
    s = lax.axis_index("s")
    base_row = (c * 16 + s) * _RPW
    pltpu.sync_copy(ones_hbm, ones_v)
    pltpu.sync_copy(zeros_hbm, hist_src.at[pl.ds(s * _RPS, _RPS)])
    pltpu.sync_copy(zeros_hbm, hist_dst.at[pl.ds(s * _RPS, _RPS)])
    plsc.subcore_barrier()

    def chunk(kk, idx_hbm, hist):
        def body(k, _):
            pltpu.sync_copy(idx_hbm.at[pl.ds(base_row + k * _IDXC, _IDXC)],
                            idx_v)
            for j in range(_IDXC):
                pltpu.sync_copy(ones_v, hist.at[idx_v.at[j]], add=True)
            return _
        return body

    lax.fori_loop(0, _RPW // _IDXC, chunk(0, src_hbm, hist_src), None)
    lax.fori_loop(0, _RPW // _IDXC, chunk(0, dst_hbm, hist_dst), None)
    plsc.subcore_barrier()
    pltpu.sync_copy(hist_src.at[pl.ds(s * _RPS, _RPS)],
                    out_hbm.at[0, c, pl.ds(s * _RPS, _RPS)])
    pltpu.sync_copy(hist_dst.at[pl.ds(s * _RPS, _RPS)],
                    out_hbm.at[1, c, pl.ds(s * _RPS, _RPS)])


@functools.partial(
    pl.kernel,
    out_type=jax.ShapeDtypeStruct((2, _NH, _LANES), jnp.float32),
    mesh=_mesh,
    scratch_types=[
        pltpu.VMEM((_IDXC, _LANES), jnp.int32),
        pltpu.VMEM((_IDXC, _LANES), jnp.int32),
        pltpu.VMEM((_LANES, _LANES), jnp.float32),
        pltpu.VMEM_SHARED((_NH, _LANES), jnp.float32),
    ],
)
def _spmm_kernel(h_hbm, src_hbm, dst_hbm, zeros_hbm, out_hbm,
                 sidx_v, didx_v, msg_v, agg):
    c = lax.axis_index("c")
    s = lax.axis_index("s")
    base_row = (c * 16 + s) * _RPW
    pltpu.sync_copy(zeros_hbm, agg.at[pl.ds(s * _RPS, _RPS)])
    plsc.subcore_barrier()

    def body(k, _):
        pltpu.sync_copy(src_hbm.at[pl.ds(base_row + k * _IDXC, _IDXC)], sidx_v)
        pltpu.sync_copy(dst_hbm.at[pl.ds(base_row + k * _IDXC, _IDXC)], didx_v)
        for j in range(_IDXC):
            pltpu.sync_copy(h_hbm.at[sidx_v.at[j]], msg_v)
            pltpu.sync_copy(msg_v, agg.at[didx_v.at[j]], add=True)
        return _

    lax.fori_loop(0, _RPW // _IDXC, body, None)
    plsc.subcore_barrier()
    pltpu.sync_copy(agg.at[pl.ds(s * _RPS, _RPS)],
                    out_hbm.at[c, pl.ds(s * _RPS, _RPS)])


_R = 400
_GRID = _N // _R


def _norm(degp_ref, a):
    d = degp_ref[a, 0, :, 0:1] + degp_ref[a, 1, :, 0:1]
    return jnp.where(d > 0.0, lax.rsqrt(d), 0.0)


def _scale_body(x_ref, degp_ref, o_ref):
    o_ref[...] = x_ref[...] * _norm(degp_ref, 0)


def _dense_body(aggp_ref, degp_ref, w1_ref, b1_ref, w2_ref, o_ref):
    agg = (aggp_ref[0] + aggp_ref[1]) * _norm(degp_ref, 1)
    h1 = jnp.dot(agg, w1_ref[...], preferred_element_type=jnp.float32)
    h1 = jnp.maximum(h1 + b1_ref[...], 0.0)
    h1 = h1 * _norm(degp_ref, 0)
    o_ref[...] = jnp.dot(h1, w2_ref[...], preferred_element_type=jnp.float32)


def _final_body(aggp_ref, degp_ref, b2_ref, o_ref):
    agg = (aggp_ref[0] + aggp_ref[1]) * _norm(degp_ref, 1)
    o_ref[...] = jnp.maximum(agg + b2_ref[...], 0.0)


_degp_spec = pl.BlockSpec((2, 2, _R, 16), lambda i: (0, 0, i, 0))
_aggp_spec = pl.BlockSpec((2, _R, _LANES), lambda i: (0, i, 0))
_row_spec = pl.BlockSpec((_R, _LANES), lambda i: (i, 0))

_scale_call = pl.pallas_call(
    _scale_body,
    grid=(_GRID,),
    in_specs=[_row_spec, _degp_spec],
    out_specs=_row_spec,
    out_shape=jax.ShapeDtypeStruct((_N, _LANES), jnp.float32),
)

_dense_call = pl.pallas_call(
    _dense_body,
    grid=(_GRID,),
    in_specs=[
        _aggp_spec,
        _degp_spec,
        pl.BlockSpec((128, 256), lambda i: (0, 0)),
        pl.BlockSpec((1, 256), lambda i: (0, 0)),
        pl.BlockSpec((256, 128), lambda i: (0, 0)),
    ],
    out_specs=_row_spec,
    out_shape=jax.ShapeDtypeStruct((_N, _LANES), jnp.float32),
)

_final_call = pl.pallas_call(
    _final_body,
    grid=(_GRID,),
    in_specs=[
        _aggp_spec,
        _degp_spec,
        pl.BlockSpec((1, 128), lambda i: (0, 0)),
    ],
    out_specs=_row_spec,
    out_shape=jax.ShapeDtypeStruct((_N, _LANES), jnp.float32),
)


def kernel(features, edge_index, W1, b1, W2, b2):
    pad = _NROWS * _LANES - _E
    lane = (jnp.arange(pad, dtype=jnp.int32) % 16)
    src = edge_index[0]
    dst = edge_index[1]
    src_deg = jnp.concatenate([src, _N + lane]).reshape(_NROWS, _LANES)
    dst_deg = jnp.concatenate([dst, _N + lane]).reshape(_NROWS, _LANES)
    src_g = jnp.concatenate([src, lane]).reshape(_NROWS, _LANES)

    ones16 = jnp.ones((_LANES, 16), jnp.float32)
    zeros16 = jnp.zeros((_RPS, 16), jnp.float32)
    zeros128 = jnp.zeros((_RPS, _LANES), jnp.float32)

    degp = _deg_kernel(src_deg, dst_deg, ones16, zeros16)
    od = jnp.zeros((_N,), jnp.float32).at[src].add(1.0)
    idg = jnp.zeros((_N,), jnp.float32).at[dst].add(1.0)
    degp = jnp.zeros((2, 2, _NH, 16), jnp.float32)
    degp = degp.at[0, 0, :_N, :].set(od[:, None])
    degp = degp.at[1, 0, :_N, :].set(idg[:, None])
    h0 = _scale_call(features, degp)
    p = _spmm_kernel(h0, src_g, dst_deg, zeros128)
    h3 = _dense_call(p, degp, W1, b1.reshape(1, -1), W2)
    q = _spmm_kernel(h3, src_g, dst_deg, zeros128)
    return _final_call(q, degp, b2.reshape(1, -1))


# speedup vs baseline: 6.5502x; 1.8290x over previous
import functools

import jax
import jax.numpy as jnp
from jax import lax
from jax.experimental import pallas as pl
from jax.experimental.pallas import tpu as pltpu
from jax.experimental.pallas import tpu_sc as plsc

_N = 10000
_E = 320000
_LANES = 128
_NW = 32
_IDXC = 16
_RPW = 80
_NROWS = _NW * _RPW
_NH = 10112
_RPS = _NH // 16

_mesh = plsc.VectorSubcoreMesh(core_axis_name="c", subcore_axis_name="s")


@functools.partial(
    pl.kernel,
    out_type=jax.ShapeDtypeStruct((2, _NH, _LANES), jnp.float32),
    mesh=_mesh,
    scratch_types=[
        pltpu.VMEM((_IDXC, _LANES), jnp.int32),
        pltpu.VMEM((_LANES, _LANES), jnp.float32),
        pltpu.VMEM_SHARED((_NH, _LANES), jnp.float32),
    ],
)
def _hist_kernel(idx_hbm, ones_hbm, zeros_hbm, out_hbm, idx_v, ones_v, agg):
    c = lax.axis_index("c")
    s = lax.axis_index("s")
    base_row = (c * 16 + s) * _RPW
    pltpu.sync_copy(ones_hbm, ones_v)
    pltpu.sync_copy(zeros_hbm, agg.at[pl.ds(s * _RPS, _RPS)])
    plsc.subcore_barrier()

    def body(k, _):
        pltpu.sync_copy(idx_hbm.at[pl.ds(base_row + k * _IDXC, _IDXC)], idx_v)
        for j in range(_IDXC):
            pltpu.sync_copy(ones_v, agg.at[idx_v.at[j]], add=True)
        return _

    lax.fori_loop(0, _RPW // _IDXC, body, None)
    plsc.subcore_barrier()
    pltpu.sync_copy(agg.at[pl.ds(s * _RPS, _RPS)],
                    out_hbm.at[c, pl.ds(s * _RPS, _RPS)])


@functools.partial(
    pl.kernel,
    out_type=jax.ShapeDtypeStruct((2, _NH, _LANES), jnp.float32),
    mesh=_mesh,
    scratch_types=[
        pltpu.VMEM((_IDXC, _LANES), jnp.int32),
        pltpu.VMEM((_IDXC, _LANES), jnp.int32),
        pltpu.VMEM((_LANES, _LANES), jnp.float32),
        pltpu.VMEM_SHARED((_NH, _LANES), jnp.float32),
    ],
)
def _spmm_kernel(h_hbm, src_hbm, dst_hbm, zeros_hbm, out_hbm,
                 sidx_v, didx_v, msg_v, agg):
    c = lax.axis_index("c")
    s = lax.axis_index("s")
    base_row = (c * 16 + s) * _RPW
    pltpu.sync_copy(zeros_hbm, agg.at[pl.ds(s * _RPS, _RPS)])
    plsc.subcore_barrier()

    def body(k, _):
        pltpu.sync_copy(src_hbm.at[pl.ds(base_row + k * _IDXC, _IDXC)], sidx_v)
        pltpu.sync_copy(dst_hbm.at[pl.ds(base_row + k * _IDXC, _IDXC)], didx_v)
        for j in range(_IDXC):
            pltpu.sync_copy(h_hbm.at[sidx_v.at[j]], msg_v)
            pltpu.sync_copy(msg_v, agg.at[didx_v.at[j]], add=True)
        return _

    lax.fori_loop(0, _RPW // _IDXC, body, None)
    plsc.subcore_barrier()
    pltpu.sync_copy(agg.at[pl.ds(s * _RPS, _RPS)],
                    out_hbm.at[c, pl.ds(s * _RPS, _RPS)])


_R = 400
_GRID = _N // _R


def _norm_b(deg_ref):
    d = deg_ref[0, :, 0:1] + deg_ref[1, :, 0:1]
    return jnp.where(d > 0.0, lax.rsqrt(d), 0.0)


def _scale_body(x_ref, degs_ref, o_ref):
    o_ref[...] = x_ref[...] * _norm_b(degs_ref)


def _dense_body(aggp_ref, degs_ref, degd_ref, w1_ref, b1_ref, w2_ref, o_ref):
    agg = (aggp_ref[0] + aggp_ref[1]) * _norm_b(degd_ref)
    h1 = jnp.dot(agg, w1_ref[...], preferred_element_type=jnp.float32)
    h1 = jnp.maximum(h1 + b1_ref[...], 0.0)
    h1 = h1 * _norm_b(degs_ref)
    o_ref[...] = jnp.dot(h1, w2_ref[...], preferred_element_type=jnp.float32)


def _final_body(aggp_ref, degd_ref, b2_ref, o_ref):
    agg = (aggp_ref[0] + aggp_ref[1]) * _norm_b(degd_ref)
    o_ref[...] = jnp.maximum(agg + b2_ref[...], 0.0)


_aggp_spec = pl.BlockSpec((2, _R, _LANES), lambda i: (0, i, 0))
_row_spec = pl.BlockSpec((_R, _LANES), lambda i: (i, 0))

_scale_call = pl.pallas_call(
    _scale_body,
    grid=(_GRID,),
    in_specs=[_row_spec, _aggp_spec],
    out_specs=_row_spec,
    out_shape=jax.ShapeDtypeStruct((_N, _LANES), jnp.float32),
)

_dense_call = pl.pallas_call(
    _dense_body,
    grid=(_GRID,),
    in_specs=[
        _aggp_spec,
        _aggp_spec,
        _aggp_spec,
        pl.BlockSpec((128, 256), lambda i: (0, 0)),
        pl.BlockSpec((1, 256), lambda i: (0, 0)),
        pl.BlockSpec((256, 128), lambda i: (0, 0)),
    ],
    out_specs=_row_spec,
    out_shape=jax.ShapeDtypeStruct((_N, _LANES), jnp.float32),
)

_final_call = pl.pallas_call(
    _final_body,
    grid=(_GRID,),
    in_specs=[
        _aggp_spec,
        _aggp_spec,
        pl.BlockSpec((1, 128), lambda i: (0, 0)),
    ],
    out_specs=_row_spec,
    out_shape=jax.ShapeDtypeStruct((_N, _LANES), jnp.float32),
)


def kernel(features, edge_index, W1, b1, W2, b2):
    pad = _NROWS * _LANES - _E
    lane = (jnp.arange(pad, dtype=jnp.int32) % 16)
    src = edge_index[0]
    dst = edge_index[1]
    src_deg = jnp.concatenate([src, _N + lane]).reshape(_NROWS, _LANES)
    dst_deg = jnp.concatenate([dst, _N + lane]).reshape(_NROWS, _LANES)
    src_g = jnp.concatenate([src, lane]).reshape(_NROWS, _LANES)

    ones128 = jnp.ones((_LANES, _LANES), jnp.float32)
    zeros128 = jnp.zeros((_RPS, _LANES), jnp.float32)

    deg_s = _hist_kernel(src_deg, ones128, zeros128)
    deg_d = _hist_kernel(dst_deg, ones128, zeros128)
    h0 = _scale_call(features, deg_s)
    p = _spmm_kernel(h0, src_g, dst_deg, zeros128)
    h3 = _dense_call(p, deg_s, deg_d, W1, b1.reshape(1, -1), W2)
    q = _spmm_kernel(h3, src_g, dst_deg, zeros128)
    return _final_call(q, deg_d, b2.reshape(1, -1))


# speedup vs baseline: 7.8617x; 1.2002x over previous
import functools

import jax
import jax.numpy as jnp
from jax import lax
from jax.experimental import pallas as pl
from jax.experimental.pallas import tpu as pltpu
from jax.experimental.pallas import tpu_sc as plsc

_N = 10000
_E = 320000
_LANES = 128
_NW = 32
_IDXC = 16
_RPW = 80
_NROWS = _NW * _RPW
_NH = 10112
_RPS = _NH // 16

_mesh = plsc.VectorSubcoreMesh(core_axis_name="c", subcore_axis_name="s")


@functools.partial(
    pl.kernel,
    out_type=jax.ShapeDtypeStruct((2, _NH, _LANES), jnp.float32),
    mesh=_mesh,
    scratch_types=[
        pltpu.VMEM((2, _IDXC, _LANES), jnp.int32),
        pltpu.VMEM((_LANES, _LANES), jnp.float32),
        pltpu.VMEM_SHARED((_NH, _LANES), jnp.float32),
        pltpu.SemaphoreType.DMA,
    ],
)
def _hist_kernel(idx_hbm, ones_hbm, zeros_hbm, out_hbm, idx_v, ones_v, agg,
                 ssem):
    c = lax.axis_index("c")
    s = lax.axis_index("s")
    base_row = (c * 16 + s) * _RPW
    pltpu.sync_copy(ones_hbm, ones_v)
    pltpu.sync_copy(zeros_hbm, agg.at[pl.ds(s * _RPS, _RPS)])
    plsc.subcore_barrier()

    def drain_one():
        pltpu.make_async_copy(ones_v, agg.at[pl.ds(0, _LANES)], ssem).wait()

    def body(k, _):
        @pl.when(k >= 2)
        def _drain():
            for _j in range(_IDXC):
                drain_one()

        q = lax.rem(k, 2)
        pltpu.sync_copy(idx_hbm.at[pl.ds(base_row + k * _IDXC, _IDXC)],
                        idx_v.at[q])
        for j in range(_IDXC):
            pltpu.async_copy(ones_v, agg.at[idx_v.at[q, j]], ssem, add=True)
        return _

    lax.fori_loop(0, _RPW // _IDXC, body, None)
    for _j in range(2 * _IDXC):
        drain_one()
    plsc.subcore_barrier()
    pltpu.sync_copy(agg.at[pl.ds(s * _RPS, _RPS)],
                    out_hbm.at[c, pl.ds(s * _RPS, _RPS)])


@functools.partial(
    pl.kernel,
    out_type=jax.ShapeDtypeStruct((2, _NH, _LANES), jnp.float32),
    mesh=_mesh,
    scratch_types=[
        pltpu.VMEM((2, _IDXC, _LANES), jnp.int32),
        pltpu.VMEM((2, _IDXC, _LANES), jnp.int32),
        pltpu.VMEM((2, _LANES, _LANES), jnp.float32),
        pltpu.VMEM_SHARED((_NH, _LANES), jnp.float32),
        pltpu.SemaphoreType.DMA,
        pltpu.SemaphoreType.DMA,
    ],
)
def _spmm_kernel(h_hbm, src_hbm, dst_hbm, zeros_hbm, out_hbm,
                 sidx_v, didx_v, msg_v, agg, gsem, ssem):
    c = lax.axis_index("c")
    s = lax.axis_index("s")
    base_row = (c * 16 + s) * _RPW
    pltpu.sync_copy(zeros_hbm, agg.at[pl.ds(s * _RPS, _RPS)])
    plsc.subcore_barrier()

    def drain_gather():
        pltpu.make_async_copy(h_hbm.at[pl.ds(0, _LANES)], msg_v.at[0],
                              gsem).wait()

    def drain_scatter():
        pltpu.make_async_copy(msg_v.at[0], agg.at[pl.ds(0, _LANES)],
                              ssem).wait()

    def body(t, _):
        ck = lax.div(t, _IDXC)
        q = lax.rem(ck, 2)
        r = lax.rem(t, _IDXC)
        b = lax.rem(t, 2)

        @pl.when(r == 0)
        def _load_idx():
            pltpu.sync_copy(src_hbm.at[pl.ds(base_row + ck * _IDXC, _IDXC)],
                            sidx_v.at[q])
            pltpu.sync_copy(dst_hbm.at[pl.ds(base_row + ck * _IDXC, _IDXC)],
                            didx_v.at[q])

        @pl.when(t >= 2)
        def _drain():
            drain_scatter()

        pltpu.async_copy(h_hbm.at[sidx_v.at[q, r]], msg_v.at[b], gsem)
        drain_gather()
        pltpu.async_copy(msg_v.at[b], agg.at[didx_v.at[q, r]], ssem, add=True)
        return _

    lax.fori_loop(0, _RPW, body, None)
    for _j in range(2):
        drain_scatter()
    plsc.subcore_barrier()
    pltpu.sync_copy(agg.at[pl.ds(s * _RPS, _RPS)],
                    out_hbm.at[c, pl.ds(s * _RPS, _RPS)])


_R = 400
_GRID = _N // _R


def _norm_b(deg_ref):
    d = deg_ref[0, :, 0:1] + deg_ref[1, :, 0:1]
    return jnp.where(d > 0.0, lax.rsqrt(d), 0.0)


def _scale_body(x_ref, degs_ref, o_ref):
    o_ref[...] = x_ref[...] * _norm_b(degs_ref)


def _dense_body(aggp_ref, degs_ref, degd_ref, w1_ref, b1_ref, w2_ref, o_ref):
    agg = (aggp_ref[0] + aggp_ref[1]) * _norm_b(degd_ref)
    h1 = jnp.dot(agg, w1_ref[...], preferred_element_type=jnp.float32)
    h1 = jnp.maximum(h1 + b1_ref[...], 0.0)
    h1 = h1 * _norm_b(degs_ref)
    o_ref[...] = jnp.dot(h1, w2_ref[...], preferred_element_type=jnp.float32)


def _final_body(aggp_ref, degd_ref, b2_ref, o_ref):
    agg = (aggp_ref[0] + aggp_ref[1]) * _norm_b(degd_ref)
    o_ref[...] = jnp.maximum(agg + b2_ref[...], 0.0)


_aggp_spec = pl.BlockSpec((2, _R, _LANES), lambda i: (0, i, 0))
_row_spec = pl.BlockSpec((_R, _LANES), lambda i: (i, 0))

_scale_call = pl.pallas_call(
    _scale_body,
    grid=(_GRID,),
    in_specs=[_row_spec, _aggp_spec],
    out_specs=_row_spec,
    out_shape=jax.ShapeDtypeStruct((_N, _LANES), jnp.float32),
)

_dense_call = pl.pallas_call(
    _dense_body,
    grid=(_GRID,),
    in_specs=[
        _aggp_spec,
        _aggp_spec,
        _aggp_spec,
        pl.BlockSpec((128, 256), lambda i: (0, 0)),
        pl.BlockSpec((1, 256), lambda i: (0, 0)),
        pl.BlockSpec((256, 128), lambda i: (0, 0)),
    ],
    out_specs=_row_spec,
    out_shape=jax.ShapeDtypeStruct((_N, _LANES), jnp.float32),
)

_final_call = pl.pallas_call(
    _final_body,
    grid=(_GRID,),
    in_specs=[
        _aggp_spec,
        _aggp_spec,
        pl.BlockSpec((1, 128), lambda i: (0, 0)),
    ],
    out_specs=_row_spec,
    out_shape=jax.ShapeDtypeStruct((_N, _LANES), jnp.float32),
)


def kernel(features, edge_index, W1, b1, W2, b2):
    pad = _NROWS * _LANES - _E
    lane = (jnp.arange(pad, dtype=jnp.int32) % 16)
    src = edge_index[0]
    dst = edge_index[1]
    src_deg = jnp.concatenate([src, _N + lane]).reshape(_NROWS, _LANES)
    dst_deg = jnp.concatenate([dst, _N + lane]).reshape(_NROWS, _LANES)
    src_g = jnp.concatenate([src, lane]).reshape(_NROWS, _LANES)

    ones128 = jnp.ones((_LANES, _LANES), jnp.float32)
    zeros128 = jnp.zeros((_RPS, _LANES), jnp.float32)

    deg_s = _hist_kernel(src_deg, ones128, zeros128)
    deg_d = _hist_kernel(dst_deg, ones128, zeros128)
    h0 = _scale_call(features, deg_s)
    p = _spmm_kernel(h0, src_g, dst_deg, zeros128)
    h3 = _dense_call(p, deg_s, deg_d, W1, b1.reshape(1, -1), W2)
    q = _spmm_kernel(h3, src_g, dst_deg, zeros128)
    return _final_call(q, deg_d, b2.reshape(1, -1))


# speedup vs baseline: 10.6296x; 1.3521x over previous
import functools

import jax
import jax.numpy as jnp
from jax import lax
from jax.experimental import pallas as pl
from jax.experimental.pallas import tpu as pltpu
from jax.experimental.pallas import tpu_sc as plsc

_N = 10000
_E = 320000
_LANES = 128
_NW = 32
_IDXC = 16
_RPW = 80
_NROWS = _NW * _RPW
_NH = 10112
_RPS = _NH // 16
_NP = 10240
_NPR = _NP // _LANES

_mesh = plsc.VectorSubcoreMesh(core_axis_name="c", subcore_axis_name="s")


@functools.partial(
    pl.kernel,
    out_type=jax.ShapeDtypeStruct((_NW, 2, _NPR, _LANES), jnp.float32),
    mesh=_mesh,
    compiler_params=pltpu.CompilerParams(needs_layout_passes=False),
    scratch_types=[
        pltpu.VMEM((_RPW, _LANES), jnp.int32),
        pltpu.VMEM((_RPW, _LANES), jnp.int32),
        pltpu.VMEM((_NPR, _LANES), jnp.float32),
        pltpu.VMEM((_NPR, _LANES), jnp.float32),
    ],
)
def _hist_kernel(sidx_hbm, didx_hbm, zeros_hbm, out_hbm,
                 sidx_v, didx_v, hsrc_v, hdst_v):
    c = lax.axis_index("c")
    s = lax.axis_index("s")
    w = c * 16 + s
    base_row = w * _RPW
    pltpu.sync_copy(sidx_hbm.at[pl.ds(base_row, _RPW)], sidx_v)
    pltpu.sync_copy(didx_hbm.at[pl.ds(base_row, _RPW)], didx_v)
    pltpu.sync_copy(zeros_hbm, hsrc_v)
    pltpu.sync_copy(zeros_hbm, hdst_v)
    ones16 = jnp.full((16,), 1.0, jnp.float32)

    def body(j, carry):
        for o in range(8):
            iv = sidx_v[j, pl.ds(o * 16, 16)]
            plsc.addupdate_scatter(
                hsrc_v, [iv >> 7, iv & 127], ones16)
            jv = didx_v[j, pl.ds(o * 16, 16)]
            plsc.addupdate_scatter(
                hdst_v, [jv >> 7, jv & 127], ones16)
        return carry

    lax.fori_loop(0, _RPW, body, None)
    pltpu.sync_copy(hsrc_v, out_hbm.at[w, 0])
    pltpu.sync_copy(hdst_v, out_hbm.at[w, 1])


@functools.partial(
    pl.kernel,
    out_type=jax.ShapeDtypeStruct((2, _NH, _LANES), jnp.float32),
    mesh=_mesh,
    scratch_types=[
        pltpu.VMEM((2, _IDXC, _LANES), jnp.int32),
        pltpu.VMEM((2, _IDXC, _LANES), jnp.int32),
        pltpu.VMEM((2, _LANES, _LANES), jnp.float32),
        pltpu.VMEM_SHARED((_NH, _LANES), jnp.float32),
        pltpu.SemaphoreType.DMA,
        pltpu.SemaphoreType.DMA,
    ],
)
def _spmm_kernel(h_hbm, src_hbm, dst_hbm, zeros_hbm, out_hbm,
                 sidx_v, didx_v, msg_v, agg, gsem, ssem):
    c = lax.axis_index("c")
    s = lax.axis_index("s")
    base_row = (c * 16 + s) * _RPW
    pltpu.sync_copy(zeros_hbm, agg.at[pl.ds(s * _RPS, _RPS)])
    plsc.subcore_barrier()

    def drain_gather():
        pltpu.make_async_copy(h_hbm.at[pl.ds(0, _LANES)], msg_v.at[0],
                              gsem).wait()

    def drain_scatter():
        pltpu.make_async_copy(msg_v.at[0], agg.at[pl.ds(0, _LANES)],
                              ssem).wait()

    def body(t, _):
        ck = lax.div(t, _IDXC)
        q = lax.rem(ck, 2)
        r = lax.rem(t, _IDXC)
        b = lax.rem(t, 2)

        @pl.when(r == 0)
        def _load_idx():
            pltpu.sync_copy(src_hbm.at[pl.ds(base_row + ck * _IDXC, _IDXC)],
                            sidx_v.at[q])
            pltpu.sync_copy(dst_hbm.at[pl.ds(base_row + ck * _IDXC, _IDXC)],
                            didx_v.at[q])

        @pl.when(t >= 2)
        def _drain():
            drain_scatter()

        pltpu.async_copy(h_hbm.at[sidx_v.at[q, r]], msg_v.at[b], gsem)
        drain_gather()
        pltpu.async_copy(msg_v.at[b], agg.at[didx_v.at[q, r]], ssem, add=True)
        return _

    lax.fori_loop(0, _RPW, body, None)
    for _j in range(2):
        drain_scatter()
    plsc.subcore_barrier()
    pltpu.sync_copy(agg.at[pl.ds(s * _RPS, _RPS)],
                    out_hbm.at[c, pl.ds(s * _RPS, _RPS)])


_R = 2048
_GRID = _NP // _R
_PBR = _R // _LANES


def _prep_body(degp_ref, o_ref):
    d = jnp.sum(degp_ref[...], axis=0)
    o_ref[...] = jnp.where(d > 0.0, lax.rsqrt(d), 0.0)


def _expand(n_ref, u_ref, m_ref):
    z = jnp.dot(u_ref[...], n_ref[...], preferred_element_type=jnp.float32,
                precision=lax.Precision.HIGHEST)
    return jnp.sum(z * m_ref[...], axis=1, keepdims=True)


def _scale_body(x_ref, ns_ref, u_ref, m_ref, o_ref):
    o_ref[...] = x_ref[...] * _expand(ns_ref, u_ref, m_ref)


def _dense_body(aggp_ref, ns_ref, nd_ref, u_ref, m_ref, w1_ref, b1_ref,
                w2_ref, o_ref):
    agg = (aggp_ref[0] + aggp_ref[1]) * _expand(nd_ref, u_ref, m_ref)
    h1 = jnp.dot(agg, w1_ref[...], preferred_element_type=jnp.float32)
    h1 = jnp.maximum(h1 + b1_ref[...], 0.0)
    h1 = h1 * _expand(ns_ref, u_ref, m_ref)
    o_ref[...] = jnp.dot(h1, w2_ref[...], preferred_element_type=jnp.float32)


def _final_body(aggp_ref, nd_ref, u_ref, m_ref, b2_ref, o_ref):
    agg = (aggp_ref[0] + aggp_ref[1]) * _expand(nd_ref, u_ref, m_ref)
    o_ref[...] = jnp.maximum(agg + b2_ref[...], 0.0)


_aggp_spec = pl.BlockSpec((2, _R, _LANES), lambda i: (0, i, 0))
_row_spec = pl.BlockSpec((_R, _LANES), lambda i: (i, 0))
_norm_spec = pl.BlockSpec((_PBR, _LANES), lambda i: (i, 0))
_u_spec = pl.BlockSpec((_R, _PBR), lambda i: (0, 0))
_m_spec = pl.BlockSpec((_R, _LANES), lambda i: (0, 0))

_prep_call = pl.pallas_call(
    _prep_body,
    grid=(1,),
    in_specs=[pl.BlockSpec((_NW, 2 * _NPR, _LANES), lambda i: (0, 0, 0))],
    out_specs=pl.BlockSpec((2 * _NPR, _LANES), lambda i: (0, 0)),
    out_shape=jax.ShapeDtypeStruct((2 * _NPR, _LANES), jnp.float32),
)

_scale_call = pl.pallas_call(
    _scale_body,
    grid=(_GRID,),
    in_specs=[_row_spec, _norm_spec, _u_spec, _m_spec],
    out_specs=_row_spec,
    out_shape=jax.ShapeDtypeStruct((_N, _LANES), jnp.float32),
)

_dense_call = pl.pallas_call(
    _dense_body,
    grid=(_GRID,),
    in_specs=[
        _aggp_spec,
        _norm_spec,
        _norm_spec,
        _u_spec,
        _m_spec,
        pl.BlockSpec((128, 256), lambda i: (0, 0)),
        pl.BlockSpec((1, 256), lambda i: (0, 0)),
        pl.BlockSpec((256, 128), lambda i: (0, 0)),
    ],
    out_specs=_row_spec,
    out_shape=jax.ShapeDtypeStruct((_N, _LANES), jnp.float32),
)

_final_call = pl.pallas_call(
    _final_body,
    grid=(_GRID,),
    in_specs=[
        _aggp_spec,
        _norm_spec,
        _u_spec,
        _m_spec,
        pl.BlockSpec((1, 128), lambda i: (0, 0)),
    ],
    out_specs=_row_spec,
    out_shape=jax.ShapeDtypeStruct((_N, _LANES), jnp.float32),
)


def kernel(features, edge_index, W1, b1, W2, b2):
    pad = _NROWS * _LANES - _E
    lane = (jnp.arange(pad, dtype=jnp.int32) % 16)
    src = edge_index[0]
    dst = edge_index[1]
    src_deg = jnp.concatenate([src, _N + lane]).reshape(_NROWS, _LANES)
    dst_deg = jnp.concatenate([dst, _N + lane]).reshape(_NROWS, _LANES)
    src_g = jnp.concatenate([src, lane]).reshape(_NROWS, _LANES)

    zeros128 = jnp.zeros((_RPS, _LANES), jnp.float32)
    zeros_np = jnp.zeros((_NPR, _LANES), jnp.float32)

    rows = jnp.arange(_R, dtype=jnp.int32)
    u_mat = (rows[:, None] // _LANES ==
             jnp.arange(_PBR, dtype=jnp.int32)[None, :]).astype(jnp.float32)
    m_mat = (rows[:, None] % _LANES ==
             jnp.arange(_LANES, dtype=jnp.int32)[None, :]).astype(jnp.float32)

    degp = _hist_kernel(src_deg, dst_deg, zeros_np)
    norms = _prep_call(degp.reshape(_NW, 2 * _NPR, _LANES))
    nsrc = norms[:_NPR]
    ndst = norms[_NPR:]

    h0 = _scale_call(features, nsrc, u_mat, m_mat)
    p = _spmm_kernel(h0, src_g, dst_deg, zeros128)
    h3 = _dense_call(p, nsrc, ndst, u_mat, m_mat, W1, b1.reshape(1, -1), W2)
    q = _spmm_kernel(h3, src_g, dst_deg, zeros128)
    return _final_call(q, ndst, u_mat, m_mat, b2.reshape(1, -1))


# speedup vs baseline: 11.9168x; 1.1211x over previous
import functools

import jax
import jax.numpy as jnp
from jax import lax
from jax.experimental import pallas as pl
from jax.experimental.pallas import tpu as pltpu
from jax.experimental.pallas import tpu_sc as plsc

_N = 10000
_E = 320000
_LANES = 128
_NW = 32
_IDXC = 4
_RPW = 80
_NROWS = _NW * _RPW
_NH = 10112
_RPS = _NH // 16
_NP = 10240
_NPR = _NP // _LANES

_mesh = plsc.VectorSubcoreMesh(core_axis_name="c", subcore_axis_name="s")


@functools.partial(
    pl.kernel,
    out_type=jax.ShapeDtypeStruct((_NW, 2, _NPR, _LANES), jnp.float32),
    mesh=_mesh,
    compiler_params=pltpu.CompilerParams(needs_layout_passes=False),
    scratch_types=[
        pltpu.VMEM((_RPW, _LANES), jnp.int32),
        pltpu.VMEM((_RPW, _LANES), jnp.int32),
        pltpu.VMEM((_NPR, _LANES), jnp.float32),
        pltpu.VMEM((_NPR, _LANES), jnp.float32),
    ],
)
def _hist_kernel(sidx_hbm, didx_hbm, zeros_hbm, out_hbm,
                 sidx_v, didx_v, hsrc_v, hdst_v):
    c = lax.axis_index("c")
    s = lax.axis_index("s")
    w = c * 16 + s
    base_row = w * _RPW
    pltpu.sync_copy(sidx_hbm.at[pl.ds(base_row, _RPW)], sidx_v)
    pltpu.sync_copy(didx_hbm.at[pl.ds(base_row, _RPW)], didx_v)
    pltpu.sync_copy(zeros_hbm, hsrc_v)
    pltpu.sync_copy(zeros_hbm, hdst_v)
    ones16 = jnp.full((16,), 1.0, jnp.float32)

    def body(j, carry):
        for o in range(8):
            iv = sidx_v[j, pl.ds(o * 16, 16)]
            plsc.addupdate_scatter(
                hsrc_v, [iv >> 7, iv & 127], ones16)
            jv = didx_v[j, pl.ds(o * 16, 16)]
            plsc.addupdate_scatter(
                hdst_v, [jv >> 7, jv & 127], ones16)
        return carry

    lax.fori_loop(0, _RPW, body, None)
    pltpu.sync_copy(hsrc_v, out_hbm.at[w, 0])
    pltpu.sync_copy(hdst_v, out_hbm.at[w, 1])


_DEPTH = 2
_GIF = 1


@functools.partial(
    pl.kernel,
    out_type=jax.ShapeDtypeStruct((2, _NH, _LANES), jnp.float32),
    mesh=_mesh,
    scratch_types=[
        pltpu.VMEM((2, _IDXC, _LANES), jnp.int32),
        pltpu.VMEM((2, _IDXC, _LANES), jnp.int32),
        pltpu.VMEM((_DEPTH, _LANES, _LANES), jnp.float32),
        pltpu.VMEM_SHARED((_NH, _LANES), jnp.float32),
        pltpu.SemaphoreType.DMA,
        pltpu.SemaphoreType.DMA,
    ],
)
def _spmm_kernel(h_hbm, src_hbm, dst_hbm, zeros_hbm, out_hbm,
                 sidx_v, didx_v, msg_v, agg, gsem, ssem):
    c = lax.axis_index("c")
    s = lax.axis_index("s")
    base_row = (c * 16 + s) * _RPW
    pltpu.sync_copy(zeros_hbm, agg.at[pl.ds(s * _RPS, _RPS)])
    plsc.subcore_barrier()

    def drain_gather():
        pltpu.make_async_copy(h_hbm.at[pl.ds(0, _LANES)], msg_v.at[0],
                              gsem).wait()

    def drain_scatter():
        pltpu.make_async_copy(msg_v.at[0], agg.at[pl.ds(0, _LANES)],
                              ssem).wait()

    def issue_scatter(g):
        ck = lax.div(g, _IDXC)
        q = lax.rem(ck, 2)
        r = lax.rem(g, _IDXC)
        pltpu.async_copy(msg_v.at[lax.rem(g, _DEPTH)],
                         agg.at[didx_v.at[q, r]], ssem, add=True)

    def body(t, _):
        ck = lax.div(t, _IDXC)
        q = lax.rem(ck, 2)
        r = lax.rem(t, _IDXC)

        @pl.when(r == 0)
        def _load_idx():
            pltpu.sync_copy(src_hbm.at[pl.ds(base_row + ck * _IDXC, _IDXC)],
                            sidx_v.at[q])
            pltpu.sync_copy(dst_hbm.at[pl.ds(base_row + ck * _IDXC, _IDXC)],
                            didx_v.at[q])

        @pl.when(t >= _DEPTH)
        def _drain_s():
            drain_scatter()

        pltpu.async_copy(h_hbm.at[sidx_v.at[q, r]],
                         msg_v.at[lax.rem(t, _DEPTH)], gsem)

        @pl.when(t >= _GIF)
        def _scatter():
            drain_gather()
            issue_scatter(t - _GIF)

        return _

    lax.fori_loop(0, _RPW, body, None)
    for g in range(_RPW - _GIF, _RPW):
        drain_gather()
        issue_scatter(g)
    for _j in range(_DEPTH):
        drain_scatter()
    plsc.subcore_barrier()
    pltpu.sync_copy(agg.at[pl.ds(s * _RPS, _RPS)],
                    out_hbm.at[c, pl.ds(s * _RPS, _RPS)])


_R = 2048
_GRID = _NP // _R
_PBR = _R // _LANES


def _prep_body(degp_ref, o_ref):
    d = jnp.sum(degp_ref[...], axis=0)
    o_ref[...] = jnp.where(d > 0.0, lax.rsqrt(d), 0.0)


def _expand(n_ref, u_ref, m_ref):
    z = jnp.dot(u_ref[...], n_ref[...], preferred_element_type=jnp.float32,
                precision=lax.Precision.HIGHEST)
    return jnp.sum(z * m_ref[...], axis=1, keepdims=True)


def _scale_body(x_ref, ns_ref, u_ref, m_ref, o_ref):
    o_ref[...] = x_ref[...] * _expand(ns_ref, u_ref, m_ref)


def _dense_body(aggp_ref, ns_ref, nd_ref, u_ref, m_ref, w1_ref, b1_ref,
                w2_ref, o_ref):
    agg = (aggp_ref[0] + aggp_ref[1]) * _expand(nd_ref, u_ref, m_ref)
    h1 = jnp.dot(agg, w1_ref[...], preferred_element_type=jnp.float32)
    h1 = jnp.maximum(h1 + b1_ref[...], 0.0)
    h1 = h1 * _expand(ns_ref, u_ref, m_ref)
    o_ref[...] = jnp.dot(h1, w2_ref[...], preferred_element_type=jnp.float32)


def _final_body(aggp_ref, nd_ref, u_ref, m_ref, b2_ref, o_ref):
    agg = (aggp_ref[0] + aggp_ref[1]) * _expand(nd_ref, u_ref, m_ref)
    o_ref[...] = jnp.maximum(agg + b2_ref[...], 0.0)


_aggp_spec = pl.BlockSpec((2, _R, _LANES), lambda i: (0, i, 0))
_row_spec = pl.BlockSpec((_R, _LANES), lambda i: (i, 0))
_norm_spec = pl.BlockSpec((_PBR, _LANES), lambda i: (i, 0))
_u_spec = pl.BlockSpec((_R, _PBR), lambda i: (0, 0))
_m_spec = pl.BlockSpec((_R, _LANES), lambda i: (0, 0))

_prep_call = pl.pallas_call(
    _prep_body,
    grid=(1,),
    in_specs=[pl.BlockSpec((_NW, 2 * _NPR, _LANES), lambda i: (0, 0, 0))],
    out_specs=pl.BlockSpec((2 * _NPR, _LANES), lambda i: (0, 0)),
    out_shape=jax.ShapeDtypeStruct((2 * _NPR, _LANES), jnp.float32),
)

_scale_call = pl.pallas_call(
    _scale_body,
    grid=(_GRID,),
    in_specs=[_row_spec, _norm_spec, _u_spec, _m_spec],
    out_specs=_row_spec,
    out_shape=jax.ShapeDtypeStruct((_N, _LANES), jnp.float32),
)

_dense_call = pl.pallas_call(
    _dense_body,
    grid=(_GRID,),
    in_specs=[
        _aggp_spec,
        _norm_spec,
        _norm_spec,
        _u_spec,
        _m_spec,
        pl.BlockSpec((128, 256), lambda i: (0, 0)),
        pl.BlockSpec((1, 256), lambda i: (0, 0)),
        pl.BlockSpec((256, 128), lambda i: (0, 0)),
    ],
    out_specs=_row_spec,
    out_shape=jax.ShapeDtypeStruct((_N, _LANES), jnp.float32),
)

_final_call = pl.pallas_call(
    _final_body,
    grid=(_GRID,),
    in_specs=[
        _aggp_spec,
        _norm_spec,
        _u_spec,
        _m_spec,
        pl.BlockSpec((1, 128), lambda i: (0, 0)),
    ],
    out_specs=_row_spec,
    out_shape=jax.ShapeDtypeStruct((_N, _LANES), jnp.float32),
)


def kernel(features, edge_index, W1, b1, W2, b2):
    pad = _NROWS * _LANES - _E
    lane = (jnp.arange(pad, dtype=jnp.int32) % 16)
    src = edge_index[0]
    dst = edge_index[1]
    src_deg = jnp.concatenate([src, _N + lane]).reshape(_NROWS, _LANES)
    dst_deg = jnp.concatenate([dst, _N + lane]).reshape(_NROWS, _LANES)
    src_g = jnp.concatenate([src, lane]).reshape(_NROWS, _LANES)

    zeros128 = jnp.zeros((_RPS, _LANES), jnp.float32)
    zeros_np = jnp.zeros((_NPR, _LANES), jnp.float32)

    rows = jnp.arange(_R, dtype=jnp.int32)
    u_mat = (rows[:, None] // _LANES ==
             jnp.arange(_PBR, dtype=jnp.int32)[None, :]).astype(jnp.float32)
    m_mat = (rows[:, None] % _LANES ==
             jnp.arange(_LANES, dtype=jnp.int32)[None, :]).astype(jnp.float32)

    degp = _hist_kernel(src_deg, dst_deg, zeros_np)
    norms = _prep_call(degp.reshape(_NW, 2 * _NPR, _LANES))
    nsrc = norms[:_NPR]
    ndst = norms[_NPR:]

    h0 = _scale_call(features, nsrc, u_mat, m_mat)
    p = _spmm_kernel(h0, src_g, dst_deg, zeros128)
    h3 = _dense_call(p, nsrc, ndst, u_mat, m_mat, W1, b1.reshape(1, -1), W2)
    q = _spmm_kernel(h3, src_g, dst_deg, zeros128)
    return _final_call(q, ndst, u_mat, m_mat, b2.reshape(1, -1))


# speedup vs baseline: 12.8637x; 1.0795x over previous
import functools

import jax
import jax.numpy as jnp
from jax import lax
from jax.experimental import pallas as pl
from jax.experimental.pallas import tpu as pltpu
from jax.experimental.pallas import tpu_sc as plsc

_N = 10000
_E = 320000
_LANES = 128
_NW = 32
_IDXC = 4
_RPW = 80
_NROWS = _NW * _RPW
_NH = 10112
_RPS = _NH // 16
_NP = 10240
_NPR = _NP // _LANES

_mesh = plsc.VectorSubcoreMesh(core_axis_name="c", subcore_axis_name="s")


@functools.partial(
    pl.kernel,
    out_type=jax.ShapeDtypeStruct((_NW, 2, _NPR, _LANES), jnp.float32),
    mesh=_mesh,
    compiler_params=pltpu.CompilerParams(needs_layout_passes=False),
    scratch_types=[
        pltpu.VMEM((_RPW, _LANES), jnp.int32),
        pltpu.VMEM((_RPW, _LANES), jnp.int32),
        pltpu.VMEM((_NPR, _LANES), jnp.float32),
        pltpu.VMEM((_NPR, _LANES), jnp.float32),
    ],
)
def _hist_kernel(sidx_hbm, didx_hbm, zeros_hbm, out_hbm,
                 sidx_v, didx_v, hsrc_v, hdst_v):
    c = lax.axis_index("c")
    s = lax.axis_index("s")
    w = c * 16 + s
    base_row = w * _RPW
    pltpu.sync_copy(sidx_hbm.at[pl.ds(base_row, _RPW)], sidx_v)
    pltpu.sync_copy(didx_hbm.at[pl.ds(base_row, _RPW)], didx_v)
    pltpu.sync_copy(zeros_hbm, hsrc_v)
    pltpu.sync_copy(zeros_hbm, hdst_v)
    ones16 = jnp.full((16,), 1.0, jnp.float32)

    def body(j, carry):
        for o in range(8):
            iv = sidx_v[j, pl.ds(o * 16, 16)]
            plsc.addupdate_scatter(
                hsrc_v, [iv >> 7, iv & 127], ones16)
            jv = didx_v[j, pl.ds(o * 16, 16)]
            plsc.addupdate_scatter(
                hdst_v, [jv >> 7, jv & 127], ones16)
        return carry

    lax.fori_loop(0, _RPW, body, None)
    pltpu.sync_copy(hsrc_v, out_hbm.at[w, 0])
    pltpu.sync_copy(hdst_v, out_hbm.at[w, 1])


_DEPTH = 5
_GIF = 4
_SE = 64
_NSTR = _RPW * 2


@functools.partial(
    pl.kernel,
    out_type=jax.ShapeDtypeStruct((2, _NH, _LANES), jnp.float32),
    mesh=_mesh,
    scratch_types=[
        pltpu.VMEM((2, _IDXC, _LANES), jnp.int32),
        pltpu.VMEM((2, _IDXC, _LANES), jnp.int32),
        pltpu.VMEM((_DEPTH, _SE, _LANES), jnp.float32),
        pltpu.VMEM_SHARED((_NH, _LANES), jnp.float32),
        pltpu.SemaphoreType.DMA,
        pltpu.SemaphoreType.DMA,
    ],
)
def _spmm_kernel(h_hbm, src_hbm, dst_hbm, zeros_hbm, out_hbm,
                 sidx_v, didx_v, msg_v, agg, gsem, ssem):
    c = lax.axis_index("c")
    s = lax.axis_index("s")
    base_row = (c * 16 + s) * _RPW
    pltpu.sync_copy(zeros_hbm, agg.at[pl.ds(s * _RPS, _RPS)])
    plsc.subcore_barrier()

    def drain_gather():
        pltpu.make_async_copy(h_hbm.at[pl.ds(0, _SE)], msg_v.at[0],
                              gsem).wait()

    def drain_scatter():
        pltpu.make_async_copy(msg_v.at[0], agg.at[pl.ds(0, _SE)],
                              ssem).wait()

    def issue_scatter(g):
        ck = lax.div(g, 2 * _IDXC)
        q = lax.rem(ck, 2)
        r = lax.rem(lax.div(g, 2), _IDXC)
        h = lax.rem(g, 2) * _SE
        pltpu.async_copy(msg_v.at[lax.rem(g, _DEPTH)],
                         agg.at[didx_v.at[q, r, pl.ds(h, _SE)]],
                         ssem, add=True)

    def body(t, _):
        ck = lax.div(t, 2 * _IDXC)
        q = lax.rem(ck, 2)
        r = lax.rem(lax.div(t, 2), _IDXC)
        h = lax.rem(t, 2) * _SE

        @pl.when(lax.rem(t, 2 * _IDXC) == 0)
        def _load_idx():
            pltpu.sync_copy(src_hbm.at[pl.ds(base_row + ck * _IDXC, _IDXC)],
                            sidx_v.at[q])
            pltpu.sync_copy(dst_hbm.at[pl.ds(base_row + ck * _IDXC, _IDXC)],
                            didx_v.at[q])

        @pl.when(t >= _DEPTH)
        def _drain_s():
            drain_scatter()

        pltpu.async_copy(h_hbm.at[sidx_v.at[q, r, pl.ds(h, _SE)]],
                         msg_v.at[lax.rem(t, _DEPTH)], gsem)

        @pl.when(t >= _GIF)
        def _scatter():
            drain_gather()
            issue_scatter(t - _GIF)

        return _

    lax.fori_loop(0, _NSTR, body, None)
    for g in range(_NSTR - _GIF, _NSTR):
        drain_gather()
        issue_scatter(g)
    for _j in range(_DEPTH):
        drain_scatter()
    plsc.subcore_barrier()
    pltpu.sync_copy(agg.at[pl.ds(s * _RPS, _RPS)],
                    out_hbm.at[c, pl.ds(s * _RPS, _RPS)])


_R = 2048
_GRID = _NP // _R
_PBR = _R // _LANES


def _prep_body(degp_ref, o_ref):
    d = jnp.sum(degp_ref[...], axis=0)
    o_ref[...] = jnp.where(d > 0.0, lax.rsqrt(d), 0.0)


def _expand(n_ref, u_ref, m_ref):
    z = jnp.dot(u_ref[...], n_ref[...], preferred_element_type=jnp.float32,
                precision=lax.Precision.HIGHEST)
    return jnp.sum(z * m_ref[...], axis=1, keepdims=True)


def _scale_body(x_ref, ns_ref, u_ref, m_ref, o_ref):
    o_ref[...] = x_ref[...] * _expand(ns_ref, u_ref, m_ref)


def _dense_body(aggp_ref, ns_ref, nd_ref, u_ref, m_ref, w1_ref, b1_ref,
                w2_ref, o_ref):
    agg = (aggp_ref[0] + aggp_ref[1]) * _expand(nd_ref, u_ref, m_ref)
    h1 = jnp.dot(agg, w1_ref[...], preferred_element_type=jnp.float32)
    h1 = jnp.maximum(h1 + b1_ref[...], 0.0)
    h1 = h1 * _expand(ns_ref, u_ref, m_ref)
    o_ref[...] = jnp.dot(h1, w2_ref[...], preferred_element_type=jnp.float32)


def _final_body(aggp_ref, nd_ref, u_ref, m_ref, b2_ref, o_ref):
    agg = (aggp_ref[0] + aggp_ref[1]) * _expand(nd_ref, u_ref, m_ref)
    o_ref[...] = jnp.maximum(agg + b2_ref[...], 0.0)


_aggp_spec = pl.BlockSpec((2, _R, _LANES), lambda i: (0, i, 0))
_row_spec = pl.BlockSpec((_R, _LANES), lambda i: (i, 0))
_norm_spec = pl.BlockSpec((_PBR, _LANES), lambda i: (i, 0))
_u_spec = pl.BlockSpec((_R, _PBR), lambda i: (0, 0))
_m_spec = pl.BlockSpec((_R, _LANES), lambda i: (0, 0))

_prep_call = pl.pallas_call(
    _prep_body,
    grid=(1,),
    in_specs=[pl.BlockSpec((_NW, 2 * _NPR, _LANES), lambda i: (0, 0, 0))],
    out_specs=pl.BlockSpec((2 * _NPR, _LANES), lambda i: (0, 0)),
    out_shape=jax.ShapeDtypeStruct((2 * _NPR, _LANES), jnp.float32),
)

_scale_call = pl.pallas_call(
    _scale_body,
    grid=(_GRID,),
    in_specs=[_row_spec, _norm_spec, _u_spec, _m_spec],
    out_specs=_row_spec,
    out_shape=jax.ShapeDtypeStruct((_N, _LANES), jnp.float32),
)

_dense_call = pl.pallas_call(
    _dense_body,
    grid=(_GRID,),
    in_specs=[
        _aggp_spec,
        _norm_spec,
        _norm_spec,
        _u_spec,
        _m_spec,
        pl.BlockSpec((128, 256), lambda i: (0, 0)),
        pl.BlockSpec((1, 256), lambda i: (0, 0)),
        pl.BlockSpec((256, 128), lambda i: (0, 0)),
    ],
    out_specs=_row_spec,
    out_shape=jax.ShapeDtypeStruct((_N, _LANES), jnp.float32),
)

_final_call = pl.pallas_call(
    _final_body,
    grid=(_GRID,),
    in_specs=[
        _aggp_spec,
        _norm_spec,
        _u_spec,
        _m_spec,
        pl.BlockSpec((1, 128), lambda i: (0, 0)),
    ],
    out_specs=_row_spec,
    out_shape=jax.ShapeDtypeStruct((_N, _LANES), jnp.float32),
)


def kernel(features, edge_index, W1, b1, W2, b2):
    pad = _NROWS * _LANES - _E
    lane = (jnp.arange(pad, dtype=jnp.int32) % 16)
    src = edge_index[0]
    dst = edge_index[1]
    src_deg = jnp.concatenate([src, _N + lane]).reshape(_NROWS, _LANES)
    dst_deg = jnp.concatenate([dst, _N + lane]).reshape(_NROWS, _LANES)
    src_g = jnp.concatenate([src, lane]).reshape(_NROWS, _LANES)

    zeros128 = jnp.zeros((_RPS, _LANES), jnp.float32)
    zeros_np = jnp.zeros((_NPR, _LANES), jnp.float32)

    rows = jnp.arange(_R, dtype=jnp.int32)
    u_mat = (rows[:, None] // _LANES ==
             jnp.arange(_PBR, dtype=jnp.int32)[None, :]).astype(jnp.float32)
    m_mat = (rows[:, None] % _LANES ==
             jnp.arange(_LANES, dtype=jnp.int32)[None, :]).astype(jnp.float32)

    degp = _hist_kernel(src_deg, dst_deg, zeros_np)
    norms = _prep_call(degp.reshape(_NW, 2 * _NPR, _LANES))
    nsrc = norms[:_NPR]
    ndst = norms[_NPR:]

    h0 = _scale_call(features, nsrc, u_mat, m_mat)
    p = _spmm_kernel(h0, src_g, dst_deg, zeros128)
    h3 = _dense_call(p, nsrc, ndst, u_mat, m_mat, W1, b1.reshape(1, -1), W2)
    q = _spmm_kernel(h3, src_g, dst_deg, zeros128)
    return _final_call(q, ndst, u_mat, m_mat, b2.reshape(1, -1))
